# Initial kernel scaffold; baseline (speedup 1.0000x reference)
import functools

import jax
import jax.numpy as jnp
from jax import lax
from jax.experimental import pallas as pl
from jax.experimental.pallas import tpu as pltpu
from jax.experimental.pallas import tpu_sc as plsc

N = 10000
E = 320000
IN = 128
H = 8
F = 16
HF = H * F
NEG = 0.2

NTILES = 32
K = 128
EDGES_PER_TILE = 10240
EP = NTILES * EDGES_PER_TILE
NCH = EDGES_PER_TILE // K
NP = 10016
ROWS_PER_TILE = NP // 16


def _leaky(v):
    return jnp.where(v > 0, v, NEG * v)


def _pre_body(x_ref, w_ref, ms_ref, md_ref, h_ref, stab_ref, dtab_ref):
    h = jnp.dot(x_ref[...], w_ref[...], preferred_element_type=jnp.float32)
    h_ref[...] = h
    als = jnp.dot(h, ms_ref[...], preferred_element_type=jnp.float32)
    ald = jnp.dot(h, md_ref[...], preferred_element_type=jnp.float32)
    es = _leaky(als + ald)
    z8 = jnp.zeros_like(als)
    stab_ref[...] = jnp.concatenate([als, z8], axis=1)
    dtab_ref[...] = jnp.concatenate([ald, z8, es, z8], axis=1)


def _tc_prologue(x, W, Ms, Md):
    return pl.pallas_call(
        _pre_body,
        out_shape=(
            jax.ShapeDtypeStruct((N, HF), jnp.float32),
            jax.ShapeDtypeStruct((N, 16), jnp.float32),
            jax.ShapeDtypeStruct((N, 32), jnp.float32),
        ),
    )(x, W, Ms, Md)


def _bcast_lane(v, j):
    idx = jnp.broadcast_to(jnp.int32(j), (16,))
    return lax.gather(
        v, idx[:, None],
        dimension_numbers=lax.GatherDimensionNumbers(
            offset_dims=(), collapsed_slice_dims=(0,), start_index_map=(0,)),
        slice_sizes=(1,),
        mode=lax.GatherScatterMode.PROMISE_IN_BOUNDS)


def _sc_edge_kernel(h_hbm, stab_hbm, dtab_hbm, srcs_hbm, dsts_hbm,
                    outp_hbm, denp_hbm,
                    idxs_v, idxd_v, hbuf, sbuf, dbuf, exbuf,
                    out_acc, den_acc):
    c = lax.axis_index("c")
    s = lax.axis_index("s")

    def _zrow(i, _):
        for j in range(HF // 16):
            hbuf[i, pl.ds(16 * j, 16)] = jnp.zeros((16,), jnp.float32)
        exbuf[i, :] = jnp.zeros((16,), jnp.float32)
        return _
    lax.fori_loop(0, K, _zrow, None)

    r0 = s * ROWS_PER_TILE
    done = 0
    for rows in (128, 128, 128, 128, ROWS_PER_TILE - 512):
        pltpu.sync_copy(hbuf.at[pl.ds(0, rows)],
                        out_acc.at[pl.ds(r0 + done, rows)])
        pltpu.sync_copy(exbuf.at[pl.ds(0, rows)],
                        den_acc.at[pl.ds(r0 + done, rows)])
        done += rows
    plsc.subcore_barrier()

    tile_base = (c * 16 + s) * EDGES_PER_TILE

    def _chunk(g, _):
        base = tile_base + g * K
        pltpu.sync_copy(srcs_hbm.at[pl.ds(base, K)], idxs_v)
        pltpu.sync_copy(dsts_hbm.at[pl.ds(base, K)], idxd_v)
        pltpu.sync_copy(h_hbm.at[idxs_v], hbuf)
        pltpu.sync_copy(stab_hbm.at[idxs_v], sbuf)
        pltpu.sync_copy(dtab_hbm.at[idxd_v], dbuf)

        def _edge(i, _):
            srow = sbuf[i, :]
            ad16 = dbuf[i, pl.ds(0, 16)]
            es16 = dbuf[i, pl.ds(16, 16)]
            t = srow + ad16
            e = jnp.where(t > 0, t, NEG * t)
            ex = jnp.exp(e - es16)
            exbuf[i, :] = ex
            for j in range(H):
                b = _bcast_lane(ex, j)
                sl = pl.ds(16 * j, 16)
                hbuf[i, sl] = hbuf[i, sl] * b
            return _
        lax.fori_loop(0, K, _edge, None)

        pltpu.sync_copy(exbuf, den_acc.at[idxd_v], add=True)
        pltpu.sync_copy(hbuf, out_acc.at[idxd_v], add=True)
        return _
    lax.fori_loop(0, NCH, _chunk, None)

    plsc.subcore_barrier()
    pltpu.sync_copy(out_acc.at[pl.ds(r0, ROWS_PER_TILE)],
                    outp_hbm.at[c, pl.ds(r0, ROWS_PER_TILE)])
    pltpu.sync_copy(den_acc.at[pl.ds(r0, ROWS_PER_TILE)],
                    denp_hbm.at[c, pl.ds(r0, ROWS_PER_TILE)])


def _sc_edge_pass(h, stab, dtab, srcs, dsts):
    mesh = plsc.VectorSubcoreMesh(core_axis_name="c", subcore_axis_name="s")
    run = functools.partial(
        pl.kernel,
        mesh=mesh,
        out_type=(
            jax.ShapeDtypeStruct((2, NP, HF), jnp.float32),
            jax.ShapeDtypeStruct((2, NP, 16), jnp.float32),
        ),
        scratch_types=[
            pltpu.VMEM((K,), jnp.int32),
            pltpu.VMEM((K,), jnp.int32),
            pltpu.VMEM((K, HF), jnp.float32),
            pltpu.VMEM((K, 16), jnp.float32),
            pltpu.VMEM((K, 32), jnp.float32),
            pltpu.VMEM((K, 16), jnp.float32),
            pltpu.VMEM_SHARED((NP, HF), jnp.float32),
            pltpu.VMEM_SHARED((NP, 16), jnp.float32),
        ],
    )(_sc_edge_kernel)
    return run(h, stab, dtab, srcs, dsts)


def _post_body(outp_ref, denp_ref, h_ref, bias_ref, gamma_ref, beta_ref,
               b128_ref, o_ref):
    acc = outp_ref[0, :N, :] + outp_ref[1, :N, :] + h_ref[...]
    den = denp_ref[0, :N, :] + denp_ref[1, :N, :] + (1.0 + 1e-16)
    dinv = 1.0 / den
    dinv128 = jnp.dot(dinv, b128_ref[...],
                      preferred_element_type=jnp.float32)
    y = acc * dinv128 + bias_ref[...]
    mean = jnp.mean(y, axis=0, keepdims=True)
    var = jnp.mean((y - mean) ** 2, axis=0, keepdims=True)
    yn = (y - mean) / jnp.sqrt(var + 1e-5) * gamma_ref[...] + beta_ref[...]
    o_ref[...] = jnp.where(yn > 0, yn, NEG * yn)


def _tc_epilogue(outp, denp, h, bias, gamma, beta, B128):
    return pl.pallas_call(
        _post_body,
        out_shape=jax.ShapeDtypeStruct((N, HF), jnp.float32),
    )(outp, denp, h, bias, gamma, beta, B128)


def kernel(x, edge_index, W, a_src, a_dst, bias, gamma, beta):
    r = jnp.arange(HF, dtype=jnp.int32)
    Ms = jnp.zeros((HF, H), jnp.float32).at[r, r // F].set(a_src.reshape(-1))
    Md = jnp.zeros((HF, H), jnp.float32).at[r, r // F].set(a_dst.reshape(-1))
    B128 = jnp.zeros((16, HF), jnp.float32).at[r // F, r].set(1.0)

    h, stab, dtab = _tc_prologue(x, W, Ms, Md)

    pad = EP - E
    srcs = jnp.concatenate([edge_index[0], jnp.zeros((pad,), jnp.int32)])
    dsts = jnp.concatenate([edge_index[1], jnp.full((pad,), N, jnp.int32)])

    outp, denp = _sc_edge_pass(h, stab, dtab, srcs, dsts)

    return _tc_epilogue(outp, denp, h, bias.reshape(1, HF),
                        gamma.reshape(1, HF), beta.reshape(1, HF), B128)


# speedup vs baseline: 41.0557x; 41.0557x over previous
import functools

import jax
import jax.numpy as jnp
from jax import lax
from jax.experimental import pallas as pl
from jax.experimental.pallas import tpu as pltpu
from jax.experimental.pallas import tpu_sc as plsc

N = 10000
E = 320000
IN = 128
H = 8
F = 16
HF = H * F
NEG = 0.2

NTILES = 32
K = 128
EDGES_PER_TILE = 10240
EP = NTILES * EDGES_PER_TILE
NCH = EDGES_PER_TILE // K
NP = 10112
ROWS_PER_TILE = NP // 16


def _leaky(v):
    return jnp.where(v > 0, v, NEG * v)


def _pre_body(x_ref, w_ref, ms_ref, md_ref, h_ref, stab_ref, dtab_ref):
    h = jnp.dot(x_ref[...], w_ref[...], preferred_element_type=jnp.float32)
    h_ref[...] = h
    als = jnp.dot(h, ms_ref[...], preferred_element_type=jnp.float32)
    ald = jnp.dot(h, md_ref[...], preferred_element_type=jnp.float32)
    es = _leaky(als + ald)
    z8 = jnp.zeros_like(als)
    stab_ref[...] = jnp.concatenate([als, z8], axis=1)
    dtab_ref[...] = jnp.concatenate([ald, z8, es, z8], axis=1)


def _tc_prologue(x, W, Ms, Md):
    return pl.pallas_call(
        _pre_body,
        out_shape=(
            jax.ShapeDtypeStruct((N, HF), jnp.float32),
            jax.ShapeDtypeStruct((N, 16), jnp.float32),
            jax.ShapeDtypeStruct((N, 32), jnp.float32),
        ),
    )(x, W, Ms, Md)


def _bcast_lane(v, j):
    idx = jnp.broadcast_to(jnp.int32(j), (16,))
    return lax.gather(
        v, idx[:, None],
        dimension_numbers=lax.GatherDimensionNumbers(
            offset_dims=(), collapsed_slice_dims=(0,), start_index_map=(0,)),
        slice_sizes=(1,),
        mode=lax.GatherScatterMode.PROMISE_IN_BOUNDS)


def _sc_edge_kernel(h_hbm, stab_hbm, dtab_hbm, srcs_hbm, dsts_hbm,
                    outp_hbm, denp_hbm,
                    idxs_v, idxd_v, hbuf, sbuf, dbuf, exbuf,
                    out_acc, den_acc):
    c = lax.axis_index("c")
    s = lax.axis_index("s")

    def _zrow(i, _):
        for j in range(HF // 16):
            hbuf[i, pl.ds(16 * j, 16)] = jnp.zeros((16,), jnp.float32)
        exbuf[i, :] = jnp.zeros((16,), jnp.float32)
        return _
    lax.fori_loop(0, K, _zrow, None)

    r0 = s * ROWS_PER_TILE
    done = 0
    for rows in (K, K, K, K, ROWS_PER_TILE - 4 * K):
        pltpu.sync_copy(hbuf.at[pl.ds(0, rows)],
                        out_acc.at[pl.ds(r0 + done, rows)])
        pltpu.sync_copy(exbuf.at[pl.ds(0, rows)],
                        den_acc.at[pl.ds(r0 + done, rows)])
        done += rows
    plsc.subcore_barrier()

    tile_base = (c * 16 + s) * EDGES_PER_TILE

    def _chunk(g, _):
        base = tile_base + g * K
        pltpu.sync_copy(srcs_hbm.at[pl.ds(base, K)], idxs_v)
        pltpu.sync_copy(dsts_hbm.at[pl.ds(base, K)], idxd_v)
        pltpu.sync_copy(h_hbm.at[idxs_v], hbuf)
        pltpu.sync_copy(stab_hbm.at[idxs_v], sbuf)
        pltpu.sync_copy(dtab_hbm.at[idxd_v], dbuf)

        def _edge(i, _):
            srow = sbuf[i, :]
            ad16 = dbuf[i, pl.ds(0, 16)]
            es16 = dbuf[i, pl.ds(16, 16)]
            t = srow + ad16
            e = jnp.where(t > 0, t, NEG * t)
            ex = jnp.exp(e - es16)
            exbuf[i, :] = ex
            for j in range(H):
                b = _bcast_lane(ex, j)
                sl = pl.ds(16 * j, 16)
                hbuf[i, sl] = hbuf[i, sl] * b
            return _
        lax.fori_loop(0, K, _edge, None)

        pltpu.sync_copy(exbuf, den_acc.at[idxd_v], add=True)
        pltpu.sync_copy(hbuf, out_acc.at[idxd_v], add=True)
        return _
    lax.fori_loop(0, NCH, _chunk, None)

    plsc.subcore_barrier()
    pltpu.sync_copy(out_acc.at[pl.ds(r0, ROWS_PER_TILE)],
                    outp_hbm.at[c, pl.ds(r0, ROWS_PER_TILE)])
    pltpu.sync_copy(den_acc.at[pl.ds(r0, ROWS_PER_TILE)],
                    denp_hbm.at[c, pl.ds(r0, ROWS_PER_TILE)])


def _sc_edge_pass(h, stab, dtab, srcs, dsts):
    mesh = plsc.VectorSubcoreMesh(core_axis_name="c", subcore_axis_name="s")
    run = functools.partial(
        pl.kernel,
        mesh=mesh,
        compiler_params=pltpu.CompilerParams(use_tc_tiling_on_sc=False),
        out_type=(
            jax.ShapeDtypeStruct((2, NP, HF), jnp.float32),
            jax.ShapeDtypeStruct((2, NP, 16), jnp.float32),
        ),
        scratch_types=[
            pltpu.VMEM((K,), jnp.int32),
            pltpu.VMEM((K,), jnp.int32),
            pltpu.VMEM((K, HF), jnp.float32),
            pltpu.VMEM((K, 16), jnp.float32),
            pltpu.VMEM((K, 32), jnp.float32),
            pltpu.VMEM((K, 16), jnp.float32),
            pltpu.VMEM_SHARED((NP, HF), jnp.float32),
            pltpu.VMEM_SHARED((NP, 16), jnp.float32),
        ],
    )(_sc_edge_kernel)
    return run(h, stab, dtab, srcs, dsts)


def _post_body(outp_ref, denp_ref, h_ref, bias_ref, gamma_ref, beta_ref,
               b128_ref, o_ref):
    acc = outp_ref[0, :N, :] + outp_ref[1, :N, :] + h_ref[...]
    den = denp_ref[0, :N, :] + denp_ref[1, :N, :] + (1.0 + 1e-16)
    dinv = 1.0 / den
    dinv128 = jnp.dot(dinv, b128_ref[...],
                      preferred_element_type=jnp.float32)
    y = acc * dinv128 + bias_ref[...]
    mean = jnp.mean(y, axis=0, keepdims=True)
    var = jnp.mean((y - mean) ** 2, axis=0, keepdims=True)
    yn = (y - mean) / jnp.sqrt(var + 1e-5) * gamma_ref[...] + beta_ref[...]
    o_ref[...] = jnp.where(yn > 0, yn, NEG * yn)


def _tc_epilogue(outp, denp, h, bias, gamma, beta, B128):
    return pl.pallas_call(
        _post_body,
        out_shape=jax.ShapeDtypeStruct((N, HF), jnp.float32),
    )(outp, denp, h, bias, gamma, beta, B128)


def kernel(x, edge_index, W, a_src, a_dst, bias, gamma, beta):
    r = jnp.arange(HF, dtype=jnp.int32)
    Ms = jnp.zeros((HF, H), jnp.float32).at[r, r // F].set(a_src.reshape(-1))
    Md = jnp.zeros((HF, H), jnp.float32).at[r, r // F].set(a_dst.reshape(-1))
    B128 = jnp.zeros((16, HF), jnp.float32).at[r // F, r].set(1.0)

    h, stab, dtab = _tc_prologue(x, W, Ms, Md)

    pad = EP - E
    srcs = jnp.concatenate([edge_index[0], jnp.zeros((pad,), jnp.int32)])
    dsts = jnp.concatenate([edge_index[1], jnp.full((pad,), N, jnp.int32)])

    outp, denp = _sc_edge_pass(h, stab, dtab, srcs, dsts)

    return _tc_epilogue(outp, denp, h, bias.reshape(1, HF),
                        gamma.reshape(1, HF), beta.reshape(1, HF), B128)
